# baseline (speedup 1.0000x reference)
import functools

import numpy as np
import jax
import jax.numpy as jnp
from jax import lax
from jax.experimental import pallas as pl
from jax.experimental.pallas import tpu as pltpu
from jax.experimental.pallas import tpu_sc as plsc

_D = 256
_SEQ = 200
_B = 1024
_NC, _NS, _L = 2, 16, 16
_NW = _NC * _NS
_BPW = _B // _NW
_PC = 16
_NPC = 13
_P0_TAIL = _SEQ - _PC
_BQ = 2
_CH = _BQ * _PC
_NCHUNK = (_BPW // _BQ) * _NPC
_NBUF = 8
_LOOK = 4
_KT = _SEQ // _L


def _positional_encoding() -> np.ndarray:
    depth_h = _D / 2
    positions = np.arange(_SEQ)[:, np.newaxis]
    depths = np.arange(depth_h)[np.newaxis, :] / depth_h
    angle_rates = 1 / 10000 ** depths
    angle_rads = positions * angle_rates
    return np.concatenate(
        [np.sin(angle_rads), np.cos(angle_rads)], axis=-1
    ).astype(np.float32)


_POS = _positional_encoding()


def _build():
    mesh = plsc.VectorSubcoreMesh(
        core_axis_name="c", subcore_axis_name="s",
        num_cores=_NC, num_subcores=_NS,
    )

    @functools.partial(
        pl.kernel,
        out_type=jax.ShapeDtypeStruct((_B, _SEQ, _D), jnp.float32),
        mesh=mesh,
        scratch_types=[
            pltpu.VMEM((_SEQ, _D), jnp.float32),
            pltpu.VMEM((_BPW // 2, _SEQ), jnp.int32),
            pltpu.VMEM((_BPW * _SEQ,), jnp.int32),
            [pltpu.VMEM((_CH, _D), jnp.float32)] * _NBUF,
            pltpu.SemaphoreType.DMA((_NBUF,)),
            pltpu.SemaphoreType.DMA((_NBUF,)),
        ],
    )
    def embed(idx_hbm, table_hbm, pos_hbm, out_hbm, pos_v, idx_stage, idx_f,
              bufs, gsem, osem):
        wid = lax.axis_index("s") * _NC + lax.axis_index("c")
        b0 = wid * _BPW

        def stage_half(h):
            pltpu.sync_copy(
                idx_hbm.at[pl.ds(b0 + h * (_BPW // 2), _BPW // 2)],
                idx_stage)

            @pl.loop(0, _BPW // 2)
            def detile(r):
                fbase = (h * (_BPW // 2) + r) * _SEQ
                for k in range(_KT):
                    idx_f[pl.ds(fbase + k * _L, _L)] = (
                        idx_stage[r, pl.ds(k * _L, _L)])
                idx_f[pl.ds(fbase + _SEQ - _L, _L)] = (
                    idx_stage[r, pl.ds(_SEQ - _L, _L)])

        def chunk_coords(c):
            bq = c // _NPC
            pc = c % _NPC
            p0 = jnp.where(pc == _NPC - 1, _P0_TAIL, pc * _PC)
            return bq * _BQ, p0

        def gather(coords, b):
            brow, p0 = coords
            for bs in range(_BQ):
                pltpu.async_copy(
                    table_hbm.at[
                        idx_f.at[pl.ds((brow + bs) * _SEQ + p0, _PC)]],
                    bufs[b].at[pl.ds(bs * _PC, _PC)],
                    gsem.at[b])

        def gather_wait(coords, b):
            brow, p0 = coords
            for bs in range(_BQ):
                pltpu.make_async_copy(
                    table_hbm.at[
                        idx_f.at[pl.ds((brow + bs) * _SEQ + p0, _PC)]],
                    bufs[b].at[pl.ds(bs * _PC, _PC)],
                    gsem.at[b]).wait()

        def writeout(coords, b):
            brow, p0 = coords
            for bs in range(_BQ):
                pltpu.async_copy(
                    bufs[b].at[pl.ds(bs * _PC, _PC)],
                    out_hbm.at[b0 + brow + bs, pl.ds(p0, _PC)],
                    osem.at[b])

        def writeout_wait(b):
            pltpu.make_async_copy(table_hbm.at[pl.ds(0, _CH)], bufs[b],
                                  osem.at[b]).wait()

        stage_half(0)
        for i in range(_LOOK):
            gather(chunk_coords(i), i)
        pltpu.sync_copy(pos_hbm, pos_v)
        stage_half(1)

        @pl.loop(0, _NCHUNK, step=_NBUF)
        def chunk_group(t):
            for b in range(_NBUF):
                c = t + b
                coords = chunk_coords(c)
                gather_wait(coords, b)

                b2 = (b + _LOOK) % _NBUF
                c2 = c + _LOOK

                @pl.when(c2 < _NCHUNK)
                def _():
                    @pl.when(c >= _NBUF - _LOOK)
                    def _():
                        writeout_wait(b2)
                    gather(chunk_coords(c2), b2)

                buf = bufs[b]
                _, p0 = coords

                @pl.loop(0, _PC)
                def pos_body(ps):
                    pvs = [pos_v[p0 + ps, pl.ds(k * _L, _L)]
                           for k in range(_D // _L)]
                    for bs in range(_BQ):
                        r = bs * _PC + ps
                        for k in range(_D // _L):
                            off = k * _L
                            buf[r, pl.ds(off, _L)] = (
                                buf[r, pl.ds(off, _L)] * 16.0 + pvs[k]
                            )

                writeout(coords, b)

        for b in range(_NBUF):
            writeout_wait(b)

    return embed


def kernel(input, table):
    idx = input.astype(jnp.int32)
    pos = jnp.asarray(_POS)
    return _build()(idx, table, pos)


# speedup vs baseline: 1.0160x; 1.0160x over previous
import functools

import numpy as np
import jax
import jax.numpy as jnp
from jax import lax
from jax.experimental import pallas as pl
from jax.experimental.pallas import tpu as pltpu
from jax.experimental.pallas import tpu_sc as plsc

_D = 256
_SEQ = 200
_B = 1024
_NC, _NS, _L = 2, 16, 16
_NW = _NC * _NS
_BPW = _B // _NW
_PC = 16
_NPC = 13
_P0_TAIL = _SEQ - _PC
_BQ = 4
_CH = _BQ * _PC
_NCHUNK = (_BPW // _BQ) * _NPC
_NBUF = 4
_KT = _SEQ // _L


def _positional_encoding() -> np.ndarray:
    depth_h = _D / 2
    positions = np.arange(_SEQ)[:, np.newaxis]
    depths = np.arange(depth_h)[np.newaxis, :] / depth_h
    angle_rates = 1 / 10000 ** depths
    angle_rads = positions * angle_rates
    return np.concatenate(
        [np.sin(angle_rads), np.cos(angle_rads)], axis=-1
    ).astype(np.float32)


_POS = _positional_encoding()


def _build():
    mesh = plsc.VectorSubcoreMesh(
        core_axis_name="c", subcore_axis_name="s",
        num_cores=_NC, num_subcores=_NS,
    )

    @functools.partial(
        pl.kernel,
        out_type=jax.ShapeDtypeStruct((_B, _SEQ, _D), jnp.float32),
        mesh=mesh,
        scratch_types=[
            pltpu.VMEM((_SEQ, _D), jnp.float32),
            pltpu.VMEM((_BPW // 2, _SEQ), jnp.int32),
            pltpu.VMEM((_NCHUNK * _CH,), jnp.int32),
            [pltpu.VMEM((_CH, _D), jnp.float32)] * _NBUF,
            pltpu.SemaphoreType.DMA((_NBUF,)),
            pltpu.SemaphoreType.DMA((_NBUF,)),
        ],
    )
    def embed(idx_hbm, table_hbm, pos_hbm, out_hbm, pos_v, idx_stage, idx_f,
              bufs, gsem, osem):
        wid = lax.axis_index("s") * _NC + lax.axis_index("c")
        b0 = wid * _BPW

        def stage_half(h):
            pltpu.sync_copy(
                idx_hbm.at[pl.ds(b0 + h * (_BPW // 2), _BPW // 2)],
                idx_stage)

            @pl.loop(0, _BPW // 2)
            def detile(r_loc):
                r = h * (_BPW // 2) + r_loc
                cbase = (r // _BQ) * _NPC * _CH + (r % _BQ) * _PC
                for k in range(_KT):
                    idx_f[pl.ds(cbase + k * _CH, _PC)] = (
                        idx_stage[r_loc, pl.ds(k * _L, _L)])
                idx_f[pl.ds(cbase + _KT * _CH, _PC)] = (
                    idx_stage[r_loc, pl.ds(_SEQ - _L, _L)])

        def chunk_coords(c):
            bq = c // _NPC
            pc = c % _NPC
            p0 = jnp.where(pc == _NPC - 1, _P0_TAIL, pc * _PC)
            return bq * _BQ, p0

        def gather(c, b):
            pltpu.async_copy(
                table_hbm.at[idx_f.at[pl.ds(c * _CH, _CH)]],
                bufs[b], gsem.at[b])

        def gather_wait(c, b):
            pltpu.make_async_copy(
                table_hbm.at[idx_f.at[pl.ds(c * _CH, _CH)]],
                bufs[b], gsem.at[b]).wait()

        def writeout(coords, b):
            brow, p0 = coords
            for bs in range(_BQ):
                pltpu.async_copy(
                    bufs[b].at[pl.ds(bs * _PC, _PC)],
                    out_hbm.at[b0 + brow + bs, pl.ds(p0, _PC)],
                    osem.at[b])

        def writeout_wait(b):
            pltpu.make_async_copy(table_hbm.at[pl.ds(0, _CH)], bufs[b],
                                  osem.at[b]).wait()

        stage_half(0)
        gather(0, 0)
        gather(1, 1)
        pltpu.sync_copy(pos_hbm, pos_v)
        stage_half(1)

        @pl.loop(0, _NCHUNK, step=_NBUF)
        def chunk_group(t):
            for b in range(_NBUF):
                c = t + b
                coords = chunk_coords(c)
                gather_wait(c, b)

                b2 = (b + 2) % _NBUF
                c2 = c + 2

                @pl.when(c2 < _NCHUNK)
                def _():
                    @pl.when(c >= 2)
                    def _():
                        writeout_wait(b2)
                    gather(c2, b2)

                buf = bufs[b]
                _, p0 = coords

                @pl.loop(0, _PC)
                def pos_body(ps):
                    pvs = [pos_v[p0 + ps, pl.ds(k * _L, _L)]
                           for k in range(_D // _L)]
                    for bs in range(_BQ):
                        r = bs * _PC + ps
                        for k in range(_D // _L):
                            off = k * _L
                            buf[r, pl.ds(off, _L)] = (
                                buf[r, pl.ds(off, _L)] * 16.0 + pvs[k]
                            )

                writeout(coords, b)

        for b in range(_NBUF):
            writeout_wait(b)

    return embed


def kernel(input, table):
    idx = input.astype(jnp.int32)
    pos = jnp.asarray(_POS)
    return _build()(idx, table, pos)


# speedup vs baseline: 1.0206x; 1.0046x over previous
import functools

import numpy as np
import jax
import jax.numpy as jnp
from jax import lax
from jax.experimental import pallas as pl
from jax.experimental.pallas import tpu as pltpu
from jax.experimental.pallas import tpu_sc as plsc

_D = 256
_SEQ = 200
_B = 1024
_NC, _NS, _L = 2, 16, 16
_NW = _NC * _NS
_BPW = _B // _NW
_PC = 16
_NPC = 13
_P0_TAIL = _SEQ - _PC
_BQ = 4
_CH = _BQ * _PC
_NCHUNK = (_BPW // _BQ) * _NPC
_NBUF = 4
_KT = _SEQ // _L


def _positional_encoding() -> np.ndarray:
    depth_h = _D / 2
    positions = np.arange(_SEQ)[:, np.newaxis]
    depths = np.arange(depth_h)[np.newaxis, :] / depth_h
    angle_rates = 1 / 10000 ** depths
    angle_rads = positions * angle_rates
    return np.concatenate(
        [np.sin(angle_rads), np.cos(angle_rads)], axis=-1
    ).astype(np.float32)


_POS = _positional_encoding()


def _build():
    mesh = plsc.VectorSubcoreMesh(
        core_axis_name="c", subcore_axis_name="s",
        num_cores=_NC, num_subcores=_NS,
    )

    @functools.partial(
        pl.kernel,
        out_type=jax.ShapeDtypeStruct((_B, _SEQ, _D), jnp.float32),
        mesh=mesh,
        scratch_types=[
            pltpu.VMEM((_SEQ, _D), jnp.float32),
            pltpu.VMEM((_BPW // 2, _SEQ), jnp.int32),
            pltpu.VMEM((_NCHUNK * _CH,), jnp.int32),
            [pltpu.VMEM((_CH, _D), jnp.float32)] * _NBUF,
            pltpu.SemaphoreType.DMA((_NBUF,)),
            pltpu.SemaphoreType.DMA((_NBUF,)),
            pltpu.SemaphoreType.DMA,
        ],
    )
    def embed(idx_hbm, table_hbm, pos_hbm, out_hbm, pos_v, idx_stage, idx_f,
              bufs, gsem, osem, psem):
        wid = lax.axis_index("s") * _NC + lax.axis_index("c")
        b0 = wid * _BPW

        def stage_half(h):
            pltpu.sync_copy(
                idx_hbm.at[pl.ds(b0 + h * (_BPW // 2), _BPW // 2)],
                idx_stage)

            @pl.loop(0, _BPW // 2)
            def detile(r_loc):
                r = h * (_BPW // 2) + r_loc
                cbase = pl.multiple_of(
                    (r // _BQ) * _NPC * _CH + (r % _BQ) * _PC, _L)
                for k in range(_KT):
                    idx_f[pl.ds(cbase + k * _CH, _PC)] = (
                        idx_stage[r_loc, pl.ds(k * _L, _L)])
                idx_f[pl.ds(cbase + _KT * _CH, _PC)] = (
                    idx_stage[r_loc, pl.ds(_SEQ - _L, _L)])

        def chunk_coords(c):
            bq = c // _NPC
            pc = c % _NPC
            p0 = jnp.where(pc == _NPC - 1, _P0_TAIL, pc * _PC)
            return bq * _BQ, p0

        def gather(c, b):
            pltpu.async_copy(
                table_hbm.at[idx_f.at[pl.ds(c * _CH, _CH)]],
                bufs[b], gsem.at[b])

        def gather_wait(c, b):
            pltpu.make_async_copy(
                table_hbm.at[idx_f.at[pl.ds(c * _CH, _CH)]],
                bufs[b], gsem.at[b]).wait()

        def writeout(coords, b):
            brow, p0 = coords
            for bs in range(_BQ):
                pltpu.async_copy(
                    bufs[b].at[pl.ds(bs * _PC, _PC)],
                    out_hbm.at[b0 + brow + bs, pl.ds(p0, _PC)],
                    osem.at[b])

        def writeout_wait(b):
            pltpu.make_async_copy(table_hbm.at[pl.ds(0, _CH)], bufs[b],
                                  osem.at[b]).wait()

        pos_dma = pltpu.async_copy(pos_hbm, pos_v, psem)
        stage_half(0)
        gather(0, 0)
        gather(1, 1)
        stage_half(1)
        pos_dma.wait()

        @pl.loop(0, _NCHUNK, step=_NBUF)
        def chunk_group(t):
            for b in range(_NBUF):
                c = t + b
                coords = chunk_coords(c)
                gather_wait(c, b)

                b2 = (b + 2) % _NBUF
                c2 = c + 2

                @pl.when(c2 < _NCHUNK)
                def _():
                    @pl.when(c >= 2)
                    def _():
                        writeout_wait(b2)
                    gather(c2, b2)

                buf = bufs[b]
                _, p0 = coords

                @pl.loop(0, _PC)
                def pos_body(ps):
                    pvs = [pos_v[p0 + ps, pl.ds(k * _L, _L)]
                           for k in range(_D // _L)]
                    for bs in range(_BQ):
                        r = bs * _PC + ps
                        for k in range(_D // _L):
                            off = k * _L
                            buf[r, pl.ds(off, _L)] = (
                                buf[r, pl.ds(off, _L)] * 16.0 + pvs[k]
                            )

                writeout(coords, b)

        for b in range(_NBUF):
            writeout_wait(b)

    return embed


def kernel(input, table):
    idx = input.astype(jnp.int32)
    pos = jnp.asarray(_POS)
    return _build()(idx, table, pos)
